# baseline (speedup 1.0000x reference)
import functools

import jax
import jax.numpy as jnp
from jax.experimental import pallas as pl
from jax.experimental.pallas import tpu as pltpu

_B, _N, _C, _K, _TOPK = 2, 2048, 1024, 8, 2
_EPS = 1e-06
_RATIO = 0.1
_LB_W = 0.01

_GATE_BN = 512
_BN = 512


def _gate_kernel(tok_ref, geno_ref, Wg_ref, bg_ref, Wgg_ref, bgg_ref,
                 wt_ref, lb_ref, cnt_ref):
    b = pl.program_id(0)
    nb = pl.program_id(1)
    nblocks = pl.num_programs(1)

    tok = tok_ref[0]
    lg = jnp.dot(tok, Wg_ref[...], preferred_element_type=jnp.float32)
    g = jnp.dot(geno_ref[0], Wgg_ref[...], preferred_element_type=jnp.float32)
    lg = lg + bg_ref[...] + _RATIO * (g + bgg_ref[...])

    iota = jax.lax.broadcasted_iota(jnp.int32, lg.shape, 1)
    m1 = jnp.max(lg, axis=1, keepdims=True)
    i1 = jnp.min(jnp.where(lg == m1, iota, _K), axis=1, keepdims=True)
    oh1 = iota == i1
    lg2 = jnp.where(oh1, jnp.float32(-1e30), lg)
    m2 = jnp.max(lg2, axis=1, keepdims=True)
    i2 = jnp.min(jnp.where(lg2 == m2, iota, _K), axis=1, keepdims=True)
    oh2 = iota == i2

    e2 = jnp.exp(m2 - m1)
    denom = 1.0 + e2
    w1 = jnp.maximum(1.0 / denom, _EPS)
    w2 = jnp.maximum(e2 / denom, _EPS)
    s = w1 + w2
    w1 = w1 / s
    w2 = w2 / s
    w = jnp.where(oh1, w1, 0.0) + jnp.where(oh2, w2, 0.0)
    wt_ref[0] = w.T

    @pl.when((b == 0) & (nb == 0))
    def _():
        cnt_ref[...] = jnp.zeros_like(cnt_ref)

    hit = oh1.astype(jnp.float32) + oh2.astype(jnp.float32)
    cnt_ref[...] += jnp.sum(hit, axis=0, keepdims=True)

    @pl.when((b == pl.num_programs(0) - 1) & (nb == nblocks - 1))
    def _():
        usage = cnt_ref[...] * (1.0 / (_B * _N))
        m = jnp.mean(usage)
        v = jnp.mean((usage - m) ** 2)
        lb_ref[...] = (_LB_W * v / (m + _EPS) ** 2).reshape(1, 1)


def _expert_kernel(wt_ref, tok_ref, W1_ref, b1_ref, W2_ref, b2_ref,
                   out_ref, hacc_ref, wacc_ref):
    nb = pl.program_id(2)
    nblocks = pl.num_programs(2)

    @pl.when(nb == 0)
    def _():
        hacc_ref[...] = jnp.zeros_like(hacc_ref)
        wacc_ref[0, 0] = 0.0

    tok = tok_ref[0].astype(jnp.bfloat16)
    h = jnp.dot(tok, W1_ref[0].astype(jnp.bfloat16),
                preferred_element_type=jnp.float32)
    h = jnp.maximum(h + b1_ref[0], 0.0)
    wv = wt_ref[0]
    hacc_ref[...] += jnp.dot(wv, h, preferred_element_type=jnp.float32)
    wacc_ref[0, 0] += jnp.sum(wv)

    @pl.when(nb == nblocks - 1)
    def _():
        ws = wacc_ref[0, 0]
        num = jnp.dot(hacc_ref[...], W2_ref[0],
                      preferred_element_type=jnp.float32) + ws * b2_ref[0]
        out_ref[0] = num / (ws + _EPS)


@jax.jit
def kernel(tokens, geno_vec, Wg, bg, Wgg, bgg, W1, b1, W2, b2):
    B, N, C, K = _B, _N, _C, _K
    gnb = N // _GATE_BN

    wt, lb, _cnt = pl.pallas_call(
        _gate_kernel,
        grid=(B, gnb),
        in_specs=[
            pl.BlockSpec((1, _GATE_BN, C), lambda b, n: (b, n, 0)),
            pl.BlockSpec((1, 1, C), lambda b, n: (b, 0, 0)),
            pl.BlockSpec((C, K), lambda b, n: (0, 0)),
            pl.BlockSpec((1, K), lambda b, n: (0, 0)),
            pl.BlockSpec((C, K), lambda b, n: (0, 0)),
            pl.BlockSpec((1, K), lambda b, n: (0, 0)),
        ],
        out_specs=[
            pl.BlockSpec((1, K, _GATE_BN), lambda b, n: (b, 0, n)),
            pl.BlockSpec((1, 1), lambda b, n: (0, 0)),
            pl.BlockSpec((1, K), lambda b, n: (0, 0)),
        ],
        out_shape=[
            jax.ShapeDtypeStruct((B, K, N), jnp.float32),
            jax.ShapeDtypeStruct((1, 1), jnp.float32),
            jax.ShapeDtypeStruct((1, K), jnp.float32),
        ],
    )(tokens, geno_vec.reshape(B, 1, C), Wg, bg.reshape(1, K),
      Wgg, bgg.reshape(1, K))

    nb2 = N // _BN
    wt2 = wt.reshape(B * K, 1, N)
    centers = pl.pallas_call(
        _expert_kernel,
        grid=(K, B, nb2),
        in_specs=[
            pl.BlockSpec((1, 1, _BN), lambda k, b, n: (b * _K + k, 0, n)),
            pl.BlockSpec((1, _BN, C), lambda k, b, n: (b, n, 0)),
            pl.BlockSpec((1, C, C), lambda k, b, n: (k, 0, 0)),
            pl.BlockSpec((1, 1, C), lambda k, b, n: (k, 0, 0)),
            pl.BlockSpec((1, C, C), lambda k, b, n: (k, 0, 0)),
            pl.BlockSpec((1, 1, C), lambda k, b, n: (k, 0, 0)),
        ],
        out_specs=pl.BlockSpec((1, 1, C), lambda k, b, n: (b * _K + k, 0, 0)),
        out_shape=jax.ShapeDtypeStruct((B * K, 1, C), jnp.float32),
        scratch_shapes=[
            pltpu.VMEM((1, C), jnp.float32),
            pltpu.SMEM((1, 1), jnp.float32),
        ],
    )(wt2, tokens, W1, b1.reshape(K, 1, C), W2, b2.reshape(K, 1, C))

    return centers.reshape(B, K, C), lb.reshape(())


# speedup vs baseline: 1.0218x; 1.0218x over previous
import functools

import jax
import jax.numpy as jnp
from jax.experimental import pallas as pl
from jax.experimental.pallas import tpu as pltpu

_B, _N, _C, _K, _TOPK = 2, 2048, 1024, 8, 2
_EPS = 1e-06
_RATIO = 0.1
_LB_W = 0.01

_GATE_BN = 512
_BN = 512


def _gate_kernel(tok_ref, geno_ref, Wg_ref, bg_ref, Wgg_ref, bgg_ref,
                 wt_ref, lb_ref, cnt_ref):
    b = pl.program_id(0)
    nb = pl.program_id(1)
    nblocks = pl.num_programs(1)

    tok = tok_ref[0]
    lg = jnp.dot(tok, Wg_ref[...], preferred_element_type=jnp.float32)
    g = jnp.dot(geno_ref[0], Wgg_ref[...], preferred_element_type=jnp.float32)
    lg = lg + bg_ref[...] + _RATIO * (g + bgg_ref[...])

    iota = jax.lax.broadcasted_iota(jnp.int32, lg.shape, 1)
    m1 = jnp.max(lg, axis=1, keepdims=True)
    i1 = jnp.min(jnp.where(lg == m1, iota, _K), axis=1, keepdims=True)
    oh1 = iota == i1
    lg2 = jnp.where(oh1, jnp.float32(-1e30), lg)
    m2 = jnp.max(lg2, axis=1, keepdims=True)
    i2 = jnp.min(jnp.where(lg2 == m2, iota, _K), axis=1, keepdims=True)
    oh2 = iota == i2

    e2 = jnp.exp(m2 - m1)
    denom = 1.0 + e2
    w1 = jnp.maximum(1.0 / denom, _EPS)
    w2 = jnp.maximum(e2 / denom, _EPS)
    s = w1 + w2
    w1 = w1 / s
    w2 = w2 / s
    w = jnp.where(oh1, w1, 0.0) + jnp.where(oh2, w2, 0.0)
    wt_ref[0] = w.T

    @pl.when((b == 0) & (nb == 0))
    def _():
        cnt_ref[...] = jnp.zeros_like(cnt_ref)

    hit = oh1.astype(jnp.float32) + oh2.astype(jnp.float32)
    cnt_ref[...] += jnp.sum(hit, axis=0, keepdims=True)

    @pl.when((b == pl.num_programs(0) - 1) & (nb == nblocks - 1))
    def _():
        usage = cnt_ref[...] * (1.0 / (_B * _N))
        m = jnp.mean(usage)
        v = jnp.mean((usage - m) ** 2)
        lb_ref[...] = (_LB_W * v / (m + _EPS) ** 2).reshape(1, 1)


def _expert_kernel(wt_ref, tok_ref, W1_ref, b1_ref, W2_ref, b2_ref,
                   out_ref, hacc_ref, wacc_ref):
    nb = pl.program_id(2)
    nblocks = pl.num_programs(2)

    @pl.when(nb == 0)
    def _():
        hacc_ref[...] = jnp.zeros_like(hacc_ref)
        wacc_ref[0, 0] = 0.0

    tok = tok_ref[0].astype(jnp.bfloat16)
    h = jnp.dot(tok, W1_ref[0].astype(jnp.bfloat16),
                preferred_element_type=jnp.float32)
    h = jnp.maximum(h + b1_ref[0], 0.0)
    wv = wt_ref[0]
    hw = h * wv.reshape(_BN, 1)
    hacc_ref[...] += jnp.sum(hw.reshape(_BN // 8, 8, _C), axis=0)
    wacc_ref[0, 0] += jnp.sum(wv)

    @pl.when(nb == nblocks - 1)
    def _():
        ws = wacc_ref[0, 0]
        hsum = jnp.sum(hacc_ref[...], axis=0, keepdims=True)
        num = jnp.dot(hsum, W2_ref[0],
                      preferred_element_type=jnp.float32) + ws * b2_ref[0]
        out_ref[0] = num / (ws + _EPS)


@jax.jit
def kernel(tokens, geno_vec, Wg, bg, Wgg, bgg, W1, b1, W2, b2):
    B, N, C, K = _B, _N, _C, _K
    gnb = N // _GATE_BN

    wt, lb, _cnt = pl.pallas_call(
        _gate_kernel,
        grid=(B, gnb),
        in_specs=[
            pl.BlockSpec((1, _GATE_BN, C), lambda b, n: (b, n, 0)),
            pl.BlockSpec((1, 1, C), lambda b, n: (b, 0, 0)),
            pl.BlockSpec((C, K), lambda b, n: (0, 0)),
            pl.BlockSpec((1, K), lambda b, n: (0, 0)),
            pl.BlockSpec((C, K), lambda b, n: (0, 0)),
            pl.BlockSpec((1, K), lambda b, n: (0, 0)),
        ],
        out_specs=[
            pl.BlockSpec((1, K, _GATE_BN), lambda b, n: (b, 0, n)),
            pl.BlockSpec((1, 1), lambda b, n: (0, 0)),
            pl.BlockSpec((1, K), lambda b, n: (0, 0)),
        ],
        out_shape=[
            jax.ShapeDtypeStruct((B, K, N), jnp.float32),
            jax.ShapeDtypeStruct((1, 1), jnp.float32),
            jax.ShapeDtypeStruct((1, K), jnp.float32),
        ],
    )(tokens, geno_vec.reshape(B, 1, C), Wg, bg.reshape(1, K),
      Wgg, bgg.reshape(1, K))

    nb2 = N // _BN
    wt2 = wt.reshape(B * K, 1, N)
    centers = pl.pallas_call(
        _expert_kernel,
        grid=(K, B, nb2),
        in_specs=[
            pl.BlockSpec((1, 1, _BN), lambda k, b, n: (b * _K + k, 0, n)),
            pl.BlockSpec((1, _BN, C), lambda k, b, n: (b, n, 0)),
            pl.BlockSpec((1, C, C), lambda k, b, n: (k, 0, 0)),
            pl.BlockSpec((1, 1, C), lambda k, b, n: (k, 0, 0)),
            pl.BlockSpec((1, C, C), lambda k, b, n: (k, 0, 0)),
            pl.BlockSpec((1, 1, C), lambda k, b, n: (k, 0, 0)),
        ],
        out_specs=pl.BlockSpec((1, 1, C), lambda k, b, n: (b * _K + k, 0, 0)),
        out_shape=jax.ShapeDtypeStruct((B * K, 1, C), jnp.float32),
        scratch_shapes=[
            pltpu.VMEM((8, C), jnp.float32),
            pltpu.SMEM((1, 1), jnp.float32),
        ],
    )(wt2, tokens, W1, b1.reshape(K, 1, C), W2, b2.reshape(K, 1, C))

    return centers.reshape(B, K, C), lb.reshape(())


# speedup vs baseline: 1.1190x; 1.0951x over previous
import functools

import jax
import jax.numpy as jnp
from jax.experimental import pallas as pl
from jax.experimental.pallas import tpu as pltpu

_B, _N, _C, _K, _TOPK = 2, 2048, 1024, 8, 2
_EPS = 1e-06
_RATIO = 0.1
_LB_W = 0.01

_GATE_BN = 512
_BN = 1024


def _gate_kernel(tok_ref, geno_ref, Wg_ref, bg_ref, Wgg_ref, bgg_ref,
                 wt_ref, lb_ref, cnt_ref):
    b = pl.program_id(0)
    nb = pl.program_id(1)
    nblocks = pl.num_programs(1)

    tok = tok_ref[0]
    lg = jnp.dot(tok, Wg_ref[...], preferred_element_type=jnp.float32)
    g = jnp.dot(geno_ref[0], Wgg_ref[...], preferred_element_type=jnp.float32)
    lg = lg + bg_ref[...] + _RATIO * (g + bgg_ref[...])

    iota = jax.lax.broadcasted_iota(jnp.int32, lg.shape, 1)
    m1 = jnp.max(lg, axis=1, keepdims=True)
    i1 = jnp.min(jnp.where(lg == m1, iota, _K), axis=1, keepdims=True)
    oh1 = iota == i1
    lg2 = jnp.where(oh1, jnp.float32(-1e30), lg)
    m2 = jnp.max(lg2, axis=1, keepdims=True)
    i2 = jnp.min(jnp.where(lg2 == m2, iota, _K), axis=1, keepdims=True)
    oh2 = iota == i2

    e2 = jnp.exp(m2 - m1)
    denom = 1.0 + e2
    w1 = jnp.maximum(1.0 / denom, _EPS)
    w2 = jnp.maximum(e2 / denom, _EPS)
    s = w1 + w2
    w1 = w1 / s
    w2 = w2 / s
    w = jnp.where(oh1, w1, 0.0) + jnp.where(oh2, w2, 0.0)
    wt_ref[0] = w.T

    @pl.when((b == 0) & (nb == 0))
    def _():
        cnt_ref[...] = jnp.zeros_like(cnt_ref)

    hit = oh1.astype(jnp.float32) + oh2.astype(jnp.float32)
    cnt_ref[...] += jnp.sum(hit, axis=0, keepdims=True)

    @pl.when((b == pl.num_programs(0) - 1) & (nb == nblocks - 1))
    def _():
        usage = cnt_ref[...] * (1.0 / (_B * _N))
        m = jnp.mean(usage)
        v = jnp.mean((usage - m) ** 2)
        lb_ref[...] = (_LB_W * v / (m + _EPS) ** 2).reshape(1, 1)


def _expert_kernel(wt_ref, tok_ref, W1_ref, b1_ref, W2_ref, b2_ref,
                   out_ref, hacc_ref, wacc_ref, w1bf_ref):
    b = pl.program_id(1)
    nb = pl.program_id(2)
    nblocks = pl.num_programs(2)

    @pl.when((b == 0) & (nb == 0))
    def _():
        w1bf_ref[...] = W1_ref[0].astype(jnp.bfloat16)

    @pl.when(nb == 0)
    def _():
        hacc_ref[...] = jnp.zeros_like(hacc_ref)
        wacc_ref[0, 0] = 0.0

    tok = tok_ref[0].astype(jnp.bfloat16)
    h = jnp.dot(tok, w1bf_ref[...],
                preferred_element_type=jnp.float32)
    h = jnp.maximum(h + b1_ref[0], 0.0)
    wv = wt_ref[0]
    hw = h * wv.reshape(_BN, 1)
    hacc_ref[...] += jnp.sum(hw.reshape(_BN // 8, 8, _C), axis=0)
    wacc_ref[0, 0] += jnp.sum(wv)

    @pl.when(nb == nblocks - 1)
    def _():
        ws = wacc_ref[0, 0]
        hsum = jnp.sum(hacc_ref[...], axis=0, keepdims=True)
        num = jnp.dot(hsum, W2_ref[0],
                      preferred_element_type=jnp.float32) + ws * b2_ref[0]
        out_ref[0] = num / (ws + _EPS)


@jax.jit
def kernel(tokens, geno_vec, Wg, bg, Wgg, bgg, W1, b1, W2, b2):
    B, N, C, K = _B, _N, _C, _K
    gnb = N // _GATE_BN

    wt, lb, _cnt = pl.pallas_call(
        _gate_kernel,
        grid=(B, gnb),
        in_specs=[
            pl.BlockSpec((1, _GATE_BN, C), lambda b, n: (b, n, 0)),
            pl.BlockSpec((1, 1, C), lambda b, n: (b, 0, 0)),
            pl.BlockSpec((C, K), lambda b, n: (0, 0)),
            pl.BlockSpec((1, K), lambda b, n: (0, 0)),
            pl.BlockSpec((C, K), lambda b, n: (0, 0)),
            pl.BlockSpec((1, K), lambda b, n: (0, 0)),
        ],
        out_specs=[
            pl.BlockSpec((1, K, _GATE_BN), lambda b, n: (b, 0, n)),
            pl.BlockSpec((1, 1), lambda b, n: (0, 0)),
            pl.BlockSpec((1, K), lambda b, n: (0, 0)),
        ],
        out_shape=[
            jax.ShapeDtypeStruct((B, K, N), jnp.float32),
            jax.ShapeDtypeStruct((1, 1), jnp.float32),
            jax.ShapeDtypeStruct((1, K), jnp.float32),
        ],
    )(tokens, geno_vec.reshape(B, 1, C), Wg, bg.reshape(1, K),
      Wgg, bgg.reshape(1, K))

    nb2 = N // _BN
    wt2 = wt.reshape(B * K, 1, N)
    centers = pl.pallas_call(
        _expert_kernel,
        grid=(K, B, nb2),
        in_specs=[
            pl.BlockSpec((1, 1, _BN), lambda k, b, n: (b * _K + k, 0, n)),
            pl.BlockSpec((1, _BN, C), lambda k, b, n: (b, n, 0)),
            pl.BlockSpec((1, C, C), lambda k, b, n: (k, 0, 0)),
            pl.BlockSpec((1, 1, C), lambda k, b, n: (k, 0, 0)),
            pl.BlockSpec((1, C, C), lambda k, b, n: (k, 0, 0)),
            pl.BlockSpec((1, 1, C), lambda k, b, n: (k, 0, 0)),
        ],
        out_specs=pl.BlockSpec((1, 1, C), lambda k, b, n: (b * _K + k, 0, 0)),
        out_shape=jax.ShapeDtypeStruct((B * K, 1, C), jnp.float32),
        scratch_shapes=[
            pltpu.VMEM((8, C), jnp.float32),
            pltpu.SMEM((1, 1), jnp.float32),
            pltpu.VMEM((C, C), jnp.bfloat16),
        ],
    )(wt2, tokens, W1, b1.reshape(K, 1, C), W2, b2.reshape(K, 1, C))

    return centers.reshape(B, K, C), lb.reshape(())


# speedup vs baseline: 1.2181x; 1.0886x over previous
import functools

import jax
import jax.numpy as jnp
from jax.experimental import pallas as pl
from jax.experimental.pallas import tpu as pltpu

_B, _N, _C, _K, _TOPK = 2, 2048, 1024, 8, 2
_EPS = 1e-06
_RATIO = 0.1
_LB_W = 0.01

_GATE_BN = 512
_BN = 2048


def _gate_kernel(tok_ref, geno_ref, Wg_ref, bg_ref, Wgg_ref, bgg_ref,
                 wt_ref, lb_ref, cnt_ref):
    b = pl.program_id(0)
    nb = pl.program_id(1)
    nblocks = pl.num_programs(1)

    tok = tok_ref[0]
    lg = jnp.dot(tok, Wg_ref[...], preferred_element_type=jnp.float32)
    g = jnp.dot(geno_ref[0], Wgg_ref[...], preferred_element_type=jnp.float32)
    lg = lg + bg_ref[...] + _RATIO * (g + bgg_ref[...])

    iota = jax.lax.broadcasted_iota(jnp.int32, lg.shape, 1)
    m1 = jnp.max(lg, axis=1, keepdims=True)
    i1 = jnp.min(jnp.where(lg == m1, iota, _K), axis=1, keepdims=True)
    oh1 = iota == i1
    lg2 = jnp.where(oh1, jnp.float32(-1e30), lg)
    m2 = jnp.max(lg2, axis=1, keepdims=True)
    i2 = jnp.min(jnp.where(lg2 == m2, iota, _K), axis=1, keepdims=True)
    oh2 = iota == i2

    e2 = jnp.exp(m2 - m1)
    denom = 1.0 + e2
    w1 = jnp.maximum(1.0 / denom, _EPS)
    w2 = jnp.maximum(e2 / denom, _EPS)
    s = w1 + w2
    w1 = w1 / s
    w2 = w2 / s
    w = jnp.where(oh1, w1, 0.0) + jnp.where(oh2, w2, 0.0)
    wt_ref[0] = w.T

    @pl.when((b == 0) & (nb == 0))
    def _():
        cnt_ref[...] = jnp.zeros_like(cnt_ref)

    hit = oh1.astype(jnp.float32) + oh2.astype(jnp.float32)
    cnt_ref[...] += jnp.sum(hit, axis=0, keepdims=True)

    @pl.when((b == pl.num_programs(0) - 1) & (nb == nblocks - 1))
    def _():
        usage = cnt_ref[...] * (1.0 / (_B * _N))
        m = jnp.mean(usage)
        v = jnp.mean((usage - m) ** 2)
        lb_ref[...] = (_LB_W * v / (m + _EPS) ** 2).reshape(1, 1)


def _expert_kernel(wt_ref, tok_ref, W1_ref, b1_ref, W2_ref, b2_ref,
                   out_ref, hacc_ref, wacc_ref, w1bf_ref):
    b = pl.program_id(1)
    nb = pl.program_id(2)
    nblocks = pl.num_programs(2)

    @pl.when((b == 0) & (nb == 0))
    def _():
        w1bf_ref[...] = W1_ref[0].astype(jnp.bfloat16)

    @pl.when(nb == 0)
    def _():
        hacc_ref[...] = jnp.zeros_like(hacc_ref)
        wacc_ref[0, 0] = 0.0

    tok = tok_ref[0].astype(jnp.bfloat16)
    h = jnp.dot(tok, w1bf_ref[...],
                preferred_element_type=jnp.float32)
    h = jnp.maximum(h + b1_ref[0], 0.0)
    wv = wt_ref[0]
    hw = h * wv.reshape(_BN, 1)
    hacc_ref[...] += jnp.sum(hw.reshape(_BN // 8, 8, _C), axis=0)
    wacc_ref[0, 0] += jnp.sum(wv)

    @pl.when(nb == nblocks - 1)
    def _():
        ws = wacc_ref[0, 0]
        hsum = jnp.sum(hacc_ref[...], axis=0, keepdims=True)
        num = jnp.dot(hsum, W2_ref[0],
                      preferred_element_type=jnp.float32) + ws * b2_ref[0]
        out_ref[0] = num / (ws + _EPS)


@jax.jit
def kernel(tokens, geno_vec, Wg, bg, Wgg, bgg, W1, b1, W2, b2):
    B, N, C, K = _B, _N, _C, _K
    gnb = N // _GATE_BN

    wt, lb, _cnt = pl.pallas_call(
        _gate_kernel,
        grid=(B, gnb),
        in_specs=[
            pl.BlockSpec((1, _GATE_BN, C), lambda b, n: (b, n, 0)),
            pl.BlockSpec((1, 1, C), lambda b, n: (b, 0, 0)),
            pl.BlockSpec((C, K), lambda b, n: (0, 0)),
            pl.BlockSpec((1, K), lambda b, n: (0, 0)),
            pl.BlockSpec((C, K), lambda b, n: (0, 0)),
            pl.BlockSpec((1, K), lambda b, n: (0, 0)),
        ],
        out_specs=[
            pl.BlockSpec((1, K, _GATE_BN), lambda b, n: (b, 0, n)),
            pl.BlockSpec((1, 1), lambda b, n: (0, 0)),
            pl.BlockSpec((1, K), lambda b, n: (0, 0)),
        ],
        out_shape=[
            jax.ShapeDtypeStruct((B, K, N), jnp.float32),
            jax.ShapeDtypeStruct((1, 1), jnp.float32),
            jax.ShapeDtypeStruct((1, K), jnp.float32),
        ],
    )(tokens, geno_vec.reshape(B, 1, C), Wg, bg.reshape(1, K),
      Wgg, bgg.reshape(1, K))

    nb2 = N // _BN
    wt2 = wt.reshape(B * K, 1, N)
    centers = pl.pallas_call(
        _expert_kernel,
        grid=(K, B, nb2),
        in_specs=[
            pl.BlockSpec((1, 1, _BN), lambda k, b, n: (b * _K + k, 0, n)),
            pl.BlockSpec((1, _BN, C), lambda k, b, n: (b, n, 0)),
            pl.BlockSpec((1, C, C), lambda k, b, n: (k, 0, 0)),
            pl.BlockSpec((1, 1, C), lambda k, b, n: (k, 0, 0)),
            pl.BlockSpec((1, C, C), lambda k, b, n: (k, 0, 0)),
            pl.BlockSpec((1, 1, C), lambda k, b, n: (k, 0, 0)),
        ],
        out_specs=pl.BlockSpec((1, 1, C), lambda k, b, n: (b * _K + k, 0, 0)),
        out_shape=jax.ShapeDtypeStruct((B * K, 1, C), jnp.float32),
        scratch_shapes=[
            pltpu.VMEM((8, C), jnp.float32),
            pltpu.SMEM((1, 1), jnp.float32),
            pltpu.VMEM((C, C), jnp.bfloat16),
        ],
    )(wt2, tokens, W1, b1.reshape(K, 1, C), W2, b2.reshape(K, 1, C))

    return centers.reshape(B, K, C), lb.reshape(())


# speedup vs baseline: 1.2475x; 1.0241x over previous
import functools

import jax
import jax.numpy as jnp
from jax.experimental import pallas as pl
from jax.experimental.pallas import tpu as pltpu

_B, _N, _C, _K, _TOPK = 2, 2048, 1024, 8, 2
_EPS = 1e-06
_RATIO = 0.1
_LB_W = 0.01

_GATE_BN = 512
_BN = 2048


def _gate_kernel(tok_ref, geno_ref, Wg_ref, bg_ref, Wgg_ref, bgg_ref,
                 wt_ref, lb_ref, cnt_ref, tbf_ref):
    b = pl.program_id(0)
    nb = pl.program_id(1)
    nblocks = pl.num_programs(1)

    tok = tok_ref[0]
    tbf_ref[0] = tok.astype(jnp.bfloat16)
    lg = jnp.dot(tok, Wg_ref[...], preferred_element_type=jnp.float32)
    g = jnp.dot(geno_ref[0], Wgg_ref[...], preferred_element_type=jnp.float32)
    lg = lg + bg_ref[...] + _RATIO * (g + bgg_ref[...])

    iota = jax.lax.broadcasted_iota(jnp.int32, lg.shape, 1)
    m1 = jnp.max(lg, axis=1, keepdims=True)
    i1 = jnp.min(jnp.where(lg == m1, iota, _K), axis=1, keepdims=True)
    oh1 = iota == i1
    lg2 = jnp.where(oh1, jnp.float32(-1e30), lg)
    m2 = jnp.max(lg2, axis=1, keepdims=True)
    i2 = jnp.min(jnp.where(lg2 == m2, iota, _K), axis=1, keepdims=True)
    oh2 = iota == i2

    e2 = jnp.exp(m2 - m1)
    denom = 1.0 + e2
    w1 = jnp.maximum(1.0 / denom, _EPS)
    w2 = jnp.maximum(e2 / denom, _EPS)
    s = w1 + w2
    w1 = w1 / s
    w2 = w2 / s
    w = jnp.where(oh1, w1, 0.0) + jnp.where(oh2, w2, 0.0)
    wt_ref[0] = w.T

    @pl.when((b == 0) & (nb == 0))
    def _():
        cnt_ref[...] = jnp.zeros_like(cnt_ref)

    hit = oh1.astype(jnp.float32) + oh2.astype(jnp.float32)
    cnt_ref[...] += jnp.sum(hit, axis=0, keepdims=True)

    @pl.when((b == pl.num_programs(0) - 1) & (nb == nblocks - 1))
    def _():
        usage = cnt_ref[...] * (1.0 / (_B * _N))
        m = jnp.mean(usage)
        v = jnp.mean((usage - m) ** 2)
        lb_ref[...] = (_LB_W * v / (m + _EPS) ** 2).reshape(1, 1)


def _expert_kernel(wt_ref, tok_ref, W1_ref, b1_ref, W2_ref, b2_ref,
                   out_ref, hacc_ref, wacc_ref, w1bf_ref):
    b = pl.program_id(1)
    nb = pl.program_id(2)
    nblocks = pl.num_programs(2)

    @pl.when((b == 0) & (nb == 0))
    def _():
        w1bf_ref[...] = W1_ref[0].astype(jnp.bfloat16)

    @pl.when(nb == 0)
    def _():
        hacc_ref[...] = jnp.zeros_like(hacc_ref)
        wacc_ref[0, 0] = 0.0

    tok = tok_ref[b, pl.ds(nb * _BN, _BN), :]
    h = jnp.dot(tok, w1bf_ref[...],
                preferred_element_type=jnp.float32)
    h = jnp.maximum(h + b1_ref[0], 0.0)
    wv = wt_ref[0]
    hw = h * wv.reshape(_BN, 1)
    hacc_ref[...] += jnp.sum(hw.reshape(_BN // 8, 8, _C), axis=0)
    wacc_ref[0, 0] += jnp.sum(wv)

    @pl.when(nb == nblocks - 1)
    def _():
        ws = wacc_ref[0, 0]
        hsum = jnp.sum(hacc_ref[...], axis=0, keepdims=True)
        num = jnp.dot(hsum, W2_ref[0],
                      preferred_element_type=jnp.float32) + ws * b2_ref[0]
        out_ref[0] = num / (ws + _EPS)


@jax.jit
def kernel(tokens, geno_vec, Wg, bg, Wgg, bgg, W1, b1, W2, b2):
    B, N, C, K = _B, _N, _C, _K
    gnb = N // _GATE_BN

    wt, lb, _cnt, tbf = pl.pallas_call(
        _gate_kernel,
        grid=(B, gnb),
        in_specs=[
            pl.BlockSpec((1, _GATE_BN, C), lambda b, n: (b, n, 0)),
            pl.BlockSpec((1, 1, C), lambda b, n: (b, 0, 0)),
            pl.BlockSpec((C, K), lambda b, n: (0, 0)),
            pl.BlockSpec((1, K), lambda b, n: (0, 0)),
            pl.BlockSpec((C, K), lambda b, n: (0, 0)),
            pl.BlockSpec((1, K), lambda b, n: (0, 0)),
        ],
        out_specs=[
            pl.BlockSpec((1, K, _GATE_BN), lambda b, n: (b, 0, n)),
            pl.BlockSpec((1, 1), lambda b, n: (0, 0)),
            pl.BlockSpec((1, K), lambda b, n: (0, 0)),
            pl.BlockSpec((1, _GATE_BN, C), lambda b, n: (b, n, 0)),
        ],
        out_shape=[
            jax.ShapeDtypeStruct((B, K, N), jnp.float32),
            jax.ShapeDtypeStruct((1, 1), jnp.float32),
            jax.ShapeDtypeStruct((1, K), jnp.float32),
            jax.ShapeDtypeStruct((B, N, C), jnp.bfloat16),
        ],
    )(tokens, geno_vec.reshape(B, 1, C), Wg, bg.reshape(1, K),
      Wgg, bgg.reshape(1, K))

    nb2 = N // _BN
    wt2 = wt.reshape(B * K, 1, N)
    centers = pl.pallas_call(
        _expert_kernel,
        grid=(K, B, nb2),
        in_specs=[
            pl.BlockSpec((1, 1, _BN), lambda k, b, n: (b * _K + k, 0, n)),
            pl.BlockSpec((B, N, C), lambda k, b, n: (0, 0, 0)),
            pl.BlockSpec((1, C, C), lambda k, b, n: (k, 0, 0)),
            pl.BlockSpec((1, 1, C), lambda k, b, n: (k, 0, 0)),
            pl.BlockSpec((1, C, C), lambda k, b, n: (k, 0, 0)),
            pl.BlockSpec((1, 1, C), lambda k, b, n: (k, 0, 0)),
        ],
        out_specs=pl.BlockSpec((1, 1, C), lambda k, b, n: (b * _K + k, 0, 0)),
        out_shape=jax.ShapeDtypeStruct((B * K, 1, C), jnp.float32),
        scratch_shapes=[
            pltpu.VMEM((8, C), jnp.float32),
            pltpu.SMEM((1, 1), jnp.float32),
            pltpu.VMEM((C, C), jnp.bfloat16),
        ],
    )(wt2, tbf, W1, b1.reshape(K, 1, C), W2, b2.reshape(K, 1, C))

    return centers.reshape(B, K, C), lb.reshape(())
